# baseline (speedup 1.0000x reference)
import functools

import jax
import jax.numpy as jnp
from jax import lax
from jax.experimental import pallas as pl
from jax.experimental.pallas import tpu as pltpu
from jax.experimental.pallas import tpu_sc as plsc

VOCAB = 1000000
DIM = 64

_NC = 2
_NS = 16
_NW = _NC * _NS

_B = 4096 * 200
_BPW = _B // _NW
_CHUNK = 1024
_NSEG = _CHUNK // 128
_NCHUNK = _BPW // _CHUNK
_IDX_ROWS_PER_CHUNK = _CHUNK // 128


def _gather_body(x_hbm, w_hbm, out_hbm, idx_v, rows_v, sem):
    wid = lax.axis_index("s") * _NC + lax.axis_index("c")
    idx_row0 = wid * (_BPW // 128)
    out_row0 = wid * _BPW

    def chunk(c, _):
        pltpu.sync_copy(
            x_hbm.at[pl.ds(idx_row0 + c * _IDX_ROWS_PER_CHUNK,
                           _IDX_ROWS_PER_CHUNK)],
            idx_v,
        )
        copies = [
            pltpu.async_copy(
                w_hbm.at[idx_v.at[j]],
                rows_v.at[pl.ds(j * 128, 128)],
                sem,
            )
            for j in range(_NSEG)
        ]
        for cp in copies:
            cp.wait()
        pltpu.sync_copy(
            rows_v,
            out_hbm.at[pl.ds(out_row0 + c * _CHUNK, _CHUNK)],
        )
        return ()

    lax.fori_loop(0, _NCHUNK, chunk, (), unroll=False)


def _gather(x2, weight):
    return pl.kernel(
        _gather_body,
        out_type=jax.ShapeDtypeStruct((_B, DIM), jnp.float32),
        mesh=plsc.VectorSubcoreMesh(core_axis_name="c", subcore_axis_name="s"),
        scratch_types=[
            pltpu.VMEM((_IDX_ROWS_PER_CHUNK, 128), jnp.int32),
            pltpu.VMEM((_CHUNK, DIM), jnp.float32),
            pltpu.SemaphoreType.DMA,
        ],
        compiler_params=pltpu.CompilerParams(use_tc_tiling_on_sc=False),
    )(x2, weight)


def kernel(x, weight):
    x2 = x.reshape(_B // 128, 128).astype(jnp.int32)
    out = _gather(x2, weight)
    return out.reshape(4096, 200, DIM)


# speedup vs baseline: 1.0155x; 1.0155x over previous
import jax
import jax.numpy as jnp
from jax import lax
from jax.experimental import pallas as pl
from jax.experimental.pallas import tpu as pltpu
from jax.experimental.pallas import tpu_sc as plsc

VOCAB = 1000000
DIM = 64

_NC = 2
_NS = 16
_NW = _NC * _NS

_B = 4096 * 200
_BPW = _B // _NW
_CHUNK = 256
_NSEG = _CHUNK // 128
_NBUF = 4
_NCHUNK = _BPW // _CHUNK
_NITER = _NCHUNK // _NBUF
_IDXROWS = _BPW // 128


def _gather_body(x_hbm, w_hbm, out_hbm, idx_v, rows, gsems, wsems):
    wid = lax.axis_index("s") * _NC + lax.axis_index("c")
    idx_row0 = wid * _IDXROWS
    out_row0 = wid * _BPW

    pltpu.sync_copy(x_hbm.at[pl.ds(idx_row0, _IDXROWS)], idx_v)

    def fire_gather(c, b):
        for j in range(_NSEG):
            pltpu.async_copy(
                w_hbm.at[idx_v.at[c * _NSEG + j]],
                rows[b].at[pl.ds(j * 128, 128)],
                gsems[b],
            )

    def wait_gather(b):
        pltpu.make_async_copy(out_hbm.at[pl.ds(0, _CHUNK)], rows[b],
                              gsems[b]).wait()

    def fire_wb(c, b):
        pltpu.async_copy(rows[b], out_hbm.at[pl.ds(out_row0 + c * _CHUNK,
                                                   _CHUNK)], wsems[b])

    def wait_wb(b):
        pltpu.make_async_copy(rows[b], out_hbm.at[pl.ds(0, _CHUNK)],
                              wsems[b]).wait()

    for b in range(_NBUF):
        fire_gather(b, b)

    def outer(g, _):
        c0 = g * _NBUF
        for b in range(_NBUF):
            wait_gather(b)
            fire_wb(c0 + b, b)
        for b in range(_NBUF):
            wait_wb(b)

            @pl.when(g < _NITER - 1)
            def _():
                fire_gather(c0 + _NBUF + b, b)

        return ()

    lax.fori_loop(0, _NITER, outer, (), unroll=False)


def _gather(x2, weight):
    return pl.kernel(
        _gather_body,
        out_type=jax.ShapeDtypeStruct((_B, DIM), jnp.float32),
        mesh=plsc.VectorSubcoreMesh(core_axis_name="c", subcore_axis_name="s"),
        scratch_types=[
            pltpu.VMEM((_IDXROWS, 128), jnp.int32),
            [pltpu.VMEM((_CHUNK, DIM), jnp.float32) for _ in range(_NBUF)],
            [pltpu.SemaphoreType.DMA for _ in range(_NBUF)],
            [pltpu.SemaphoreType.DMA for _ in range(_NBUF)],
        ],
        compiler_params=pltpu.CompilerParams(use_tc_tiling_on_sc=False),
    )(x2, weight)


def kernel(x, weight):
    x2 = x.reshape(_B // 128, 128).astype(jnp.int32)
    out = _gather(x2, weight)
    return out.reshape(4096, 200, DIM)
